# baseline (speedup 1.0000x reference)
import jax
import jax.numpy as jnp
from jax import lax
from jax.experimental import pallas as pl
from jax.experimental.pallas import tpu as pltpu
from jax.experimental.pallas import tpu_sc as plsc

N_NODES = 10000
N_EDGES = 320000
D = 128
CNT = 16
CHUNK = 50
N_CHUNKS = N_EDGES // CHUNK
NC = 2
NS = 16
NW = NC * NS
TILE_ITERS = N_CHUNKS // NW
STRIPE = 624
LAST_STRIPE = N_NODES - 15 * STRIPE
NBUF = 5


def _sc_body(idx_hbm, x_hbm, sum_out, cnt_out,
             idx_v, rows_v, ones_v, zbuf, zcnt, acc_sh, cnt_sh, sems, zsem):
    c = lax.axis_index("c")
    s = lax.axis_index("s")
    wid = s * NC + c

    zrow = jnp.zeros((16,), jnp.float32)
    orow = jnp.ones((16,), jnp.float32)
    for r in range(16):
        for g in range(D // 16):
            zbuf[r, pl.ds(g * 16, 16)] = zrow
        zcnt[r, pl.ds(0, 16)] = zrow
    for r in range(CHUNK):
        ones_v[r, pl.ds(0, 16)] = orow

    n_z = jnp.where(s == NS - 1, LAST_STRIPE // 16, STRIPE // 16)

    def zero_fire(j, base):
        pltpu.async_copy(zbuf, acc_sh.at[pl.ds(base + j * 16, 16)], zsem)
        pltpu.async_copy(zcnt, cnt_sh.at[pl.ds(base + j * 16, 16)], zsem)
        return base

    def zero_drain(j, base):
        pltpu.make_async_copy(zbuf, acc_sh.at[pl.ds(base, 16)], zsem).wait()
        pltpu.make_async_copy(zcnt, cnt_sh.at[pl.ds(base, 16)], zsem).wait()
        return base

    lax.fori_loop(0, n_z, zero_fire, s * STRIPE, unroll=False)
    lax.fori_loop(0, n_z, zero_drain, s * STRIPE, unroll=False)

    plsc.subcore_barrier()

    def valid(u):
        return jnp.logical_and(u >= 0, u < TILE_ITERS)

    def idx_issue(t, k):
        chunk = t * NW + wid
        pltpu.async_copy(idx_hbm.at[chunk], idx_v.at[k], sems.at[k])

    def gather_issue(t, k):
        chunk = t * NW + wid
        pltpu.make_async_copy(idx_hbm.at[chunk], idx_v.at[k],
                              sems.at[k]).wait()
        pltpu.async_copy(x_hbm.at[idx_v.at[k, 0]], rows_v.at[k], sems.at[k])

    def scatter_issue(k):
        pltpu.make_async_copy(x_hbm.at[idx_v.at[k, 0]], rows_v.at[k],
                              sems.at[k]).wait()
        pltpu.async_copy(rows_v.at[k], acc_sh.at[idx_v.at[k, 1]], sems.at[k],
                         add=True)
        pltpu.async_copy(ones_v, cnt_sh.at[idx_v.at[k, 1]], sems.at[k],
                         add=True)

    def drain(k):
        pltpu.make_async_copy(rows_v.at[k], acc_sh.at[idx_v.at[k, 1]],
                              sems.at[k]).wait()
        pltpu.make_async_copy(ones_v, cnt_sh.at[idx_v.at[k, 1]],
                              sems.at[k]).wait()

    def pipe_step(j, carry):
        for k in range(NBUF):
            t = j * NBUF + k

            @pl.when(valid(t - 1))
            def _(t=t, k=k):
                gather_issue(t - 1, (k + NBUF - 1) % NBUF)

            @pl.when(valid(t - 5))
            def _(k=k):
                drain(k)

            @pl.when(valid(t))
            def _(t=t, k=k):
                idx_issue(t, k)

            @pl.when(valid(t - 3))
            def _(k=k):
                scatter_issue((k + 2) % NBUF)
        return carry

    n_steps = TILE_ITERS + 5
    lax.fori_loop(0, (n_steps + NBUF - 1) // NBUF, pipe_step, 0,
                  unroll=False)

    plsc.subcore_barrier()

    @pl.when(s < NS - 1)
    def _():
        base = s * STRIPE
        pltpu.sync_copy(acc_sh.at[pl.ds(base, STRIPE)],
                        sum_out.at[c, pl.ds(base, STRIPE)])
        pltpu.sync_copy(cnt_sh.at[pl.ds(base, STRIPE)],
                        cnt_out.at[c, pl.ds(base, STRIPE)])

    @pl.when(s == NS - 1)
    def _():
        base = (NS - 1) * STRIPE
        pltpu.sync_copy(acc_sh.at[pl.ds(base, LAST_STRIPE)],
                        sum_out.at[c, pl.ds(base, LAST_STRIPE)])
        pltpu.sync_copy(cnt_sh.at[pl.ds(base, LAST_STRIPE)],
                        cnt_out.at[c, pl.ds(base, LAST_STRIPE)])


@jax.jit
def _sc_aggregate(idx2d, x):
    mesh = plsc.VectorSubcoreMesh(core_axis_name="c", subcore_axis_name="s")
    return pl.kernel(
        _sc_body,
        out_type=[
            jax.ShapeDtypeStruct((NC, N_NODES, D), jnp.float32),
            jax.ShapeDtypeStruct((NC, N_NODES, CNT), jnp.float32),
        ],
        mesh=mesh,
        scratch_types=[
            pltpu.VMEM((NBUF, 2, CHUNK), jnp.int32),
            pltpu.VMEM((NBUF, CHUNK, D), jnp.float32),
            pltpu.VMEM((CHUNK, CNT), jnp.float32),
            pltpu.VMEM((16, D), jnp.float32),
            pltpu.VMEM((16, CNT), jnp.float32),
            pltpu.VMEM_SHARED((N_NODES, D), jnp.float32),
            pltpu.VMEM_SHARED((N_NODES, CNT), jnp.float32),
            pltpu.SemaphoreType.DMA((NBUF,)),
            pltpu.SemaphoreType.DMA,
        ],
        compiler_params=pltpu.CompilerParams(use_tc_tiling_on_sc=False),
    )(idx2d, x)


def _tc_xr_body(x_ref, wr_ref, bl_ref, xr_ref):
    xr_ref[...] = lax.dot_general(
        x_ref[...], wr_ref[...],
        (((1,), (1,)), ((), ())),
        precision=lax.Precision.HIGHEST,
        preferred_element_type=jnp.float32) + bl_ref[...]


def _tc_fused_body(sum_ref, cnt_ref, xr_ref, wl_ref, g_ref, bt_ref, o_ref):
    summed = sum_ref[0] + sum_ref[1]
    counts = cnt_ref[0, :, 0:1] + cnt_ref[1, :, 0:1]
    mean = summed / jnp.maximum(counts, 1.0)
    out = lax.dot_general(mean, wl_ref[...],
                          (((1,), (1,)), ((), ())),
                          precision=lax.Precision.HIGHEST,
                          preferred_element_type=jnp.float32) + xr_ref[...]
    nrm = jnp.sqrt(jnp.sum(out * out, axis=-1, keepdims=True))
    out = out / jnp.maximum(nrm, 1e-12)
    mu = jnp.sum(out, axis=0, keepdims=True) * (1.0 / N_NODES)
    ex2 = jnp.sum(out * out, axis=0, keepdims=True) * (1.0 / N_NODES)
    var = ex2 - mu * mu
    out = (out - mu) / jnp.sqrt(var + 1e-5) * g_ref[...] + bt_ref[...]
    o_ref[...] = jnp.maximum(out, 0.0)


@jax.jit
def _tc_xr(x, W_r, b_l):
    return pl.pallas_call(
        _tc_xr_body,
        out_shape=jax.ShapeDtypeStruct((N_NODES, D), jnp.float32),
    )(x, W_r, b_l.reshape(1, D))


@jax.jit
def _tc_finish(sum_parts, cnt_parts, xr, W_l, gamma, beta):
    return pl.pallas_call(
        _tc_fused_body,
        out_shape=jax.ShapeDtypeStruct((N_NODES, D), jnp.float32),
    )(sum_parts, cnt_parts, xr, W_l, gamma.reshape(1, D),
      beta.reshape(1, D))


def kernel(x, edge_index, W_l, b_l, W_r, gamma, beta):
    idx2d = jnp.transpose(
        edge_index.astype(jnp.int32).reshape(2, N_CHUNKS, CHUNK), (1, 0, 2))
    xr = _tc_xr(x, W_r, b_l)
    sum_parts, cnt_parts = _sc_aggregate(idx2d, x)
    return _tc_finish(sum_parts, cnt_parts, xr, W_l, gamma, beta)


# speedup vs baseline: 1.2291x; 1.2291x over previous
import jax
import jax.numpy as jnp
from jax import lax
from jax.experimental import pallas as pl
from jax.experimental.pallas import tpu as pltpu
from jax.experimental.pallas import tpu_sc as plsc

N_NODES = 10000
N_EDGES = 320000
D = 128
CNT = 16
CHUNK = 80
N_CHUNKS = N_EDGES // CHUNK
NC = 2
NS = 16
NW = NC * NS
TILE_ITERS = N_CHUNKS // NW
STRIPE = 624
LAST_STRIPE = N_NODES - 15 * STRIPE
NBUF = 3


def _sc_body(idx_hbm, x_hbm, sum_out, cnt_out,
             idx_v, rows_v, ones_v, zbuf, zcnt, acc_sh, cnt_sh, sems, zsem):
    c = lax.axis_index("c")
    s = lax.axis_index("s")
    wid = s * NC + c

    zrow = jnp.zeros((16,), jnp.float32)
    orow = jnp.ones((16,), jnp.float32)
    for r in range(16):
        for g in range(D // 16):
            zbuf[r, pl.ds(g * 16, 16)] = zrow
        zcnt[r, pl.ds(0, 16)] = zrow
    for r in range(CHUNK):
        ones_v[r, pl.ds(0, 16)] = orow

    n_z = jnp.where(s == NS - 1, LAST_STRIPE // 16, STRIPE // 16)

    def zero_fire(j, base):
        pltpu.async_copy(zbuf, acc_sh.at[pl.ds(base + j * 16, 16)], zsem)
        pltpu.async_copy(zcnt, cnt_sh.at[pl.ds(base + j * 16, 16)], zsem)
        return base

    def zero_drain(j, base):
        pltpu.make_async_copy(zbuf, acc_sh.at[pl.ds(base, 16)], zsem).wait()
        pltpu.make_async_copy(zcnt, cnt_sh.at[pl.ds(base, 16)], zsem).wait()
        return base

    lax.fori_loop(0, n_z, zero_fire, s * STRIPE, unroll=False)
    lax.fori_loop(0, n_z, zero_drain, s * STRIPE, unroll=False)

    plsc.subcore_barrier()

    def valid(u):
        return jnp.logical_and(u >= 0, u < TILE_ITERS)

    def idx_issue(t, k):
        chunk = t * NW + wid
        pltpu.async_copy(idx_hbm.at[chunk], idx_v.at[k], sems.at[k])

    def gather_issue(t, k):
        chunk = t * NW + wid
        pltpu.make_async_copy(idx_hbm.at[chunk], idx_v.at[k],
                              sems.at[k]).wait()
        pltpu.async_copy(x_hbm.at[idx_v.at[k, 0]], rows_v.at[k], sems.at[k])

    def scatter_issue(k):
        pltpu.make_async_copy(x_hbm.at[idx_v.at[k, 0]], rows_v.at[k],
                              sems.at[k]).wait()
        pltpu.async_copy(rows_v.at[k], acc_sh.at[idx_v.at[k, 1]], sems.at[k],
                         add=True)
        pltpu.async_copy(ones_v, cnt_sh.at[idx_v.at[k, 1]], sems.at[k],
                         add=True)

    def drain(k):
        pltpu.make_async_copy(rows_v.at[k], acc_sh.at[idx_v.at[k, 1]],
                              sems.at[k]).wait()
        pltpu.make_async_copy(ones_v, cnt_sh.at[idx_v.at[k, 1]],
                              sems.at[k]).wait()

    def pipe_step(j, carry):
        for k in range(NBUF):
            t = j * NBUF + k

            @pl.when(valid(t - 1))
            def _(t=t, k=k):
                gather_issue(t - 1, (k + 2) % NBUF)

            @pl.when(valid(t - 3))
            def _(k=k):
                drain(k)

            @pl.when(valid(t))
            def _(t=t, k=k):
                idx_issue(t, k)

            @pl.when(valid(t - 2))
            def _(k=k):
                scatter_issue((k + 1) % NBUF)
        return carry

    n_steps = TILE_ITERS + 3
    lax.fori_loop(0, (n_steps + NBUF - 1) // NBUF, pipe_step, 0,
                  unroll=False)

    plsc.subcore_barrier()

    @pl.when(s < NS - 1)
    def _():
        base = s * STRIPE
        pltpu.sync_copy(acc_sh.at[pl.ds(base, STRIPE)],
                        sum_out.at[c, pl.ds(base, STRIPE)])
        pltpu.sync_copy(cnt_sh.at[pl.ds(base, STRIPE)],
                        cnt_out.at[c, pl.ds(base, STRIPE)])

    @pl.when(s == NS - 1)
    def _():
        base = (NS - 1) * STRIPE
        pltpu.sync_copy(acc_sh.at[pl.ds(base, LAST_STRIPE)],
                        sum_out.at[c, pl.ds(base, LAST_STRIPE)])
        pltpu.sync_copy(cnt_sh.at[pl.ds(base, LAST_STRIPE)],
                        cnt_out.at[c, pl.ds(base, LAST_STRIPE)])


@jax.jit
def _sc_aggregate(idx2d, x):
    mesh = plsc.VectorSubcoreMesh(core_axis_name="c", subcore_axis_name="s")
    return pl.kernel(
        _sc_body,
        out_type=[
            jax.ShapeDtypeStruct((NC, N_NODES, D), jnp.float32),
            jax.ShapeDtypeStruct((NC, N_NODES, CNT), jnp.float32),
        ],
        mesh=mesh,
        scratch_types=[
            pltpu.VMEM((NBUF, 2, CHUNK), jnp.int32),
            pltpu.VMEM((NBUF, CHUNK, D), jnp.float32),
            pltpu.VMEM((CHUNK, CNT), jnp.float32),
            pltpu.VMEM((16, D), jnp.float32),
            pltpu.VMEM((16, CNT), jnp.float32),
            pltpu.VMEM_SHARED((N_NODES, D), jnp.float32),
            pltpu.VMEM_SHARED((N_NODES, CNT), jnp.float32),
            pltpu.SemaphoreType.DMA((NBUF,)),
            pltpu.SemaphoreType.DMA,
        ],
        compiler_params=pltpu.CompilerParams(use_tc_tiling_on_sc=False),
    )(idx2d, x)


def _tc_xr_body(x_ref, wr_ref, bl_ref, xr_ref):
    xr_ref[...] = lax.dot_general(
        x_ref[...], wr_ref[...],
        (((1,), (1,)), ((), ())),
        precision=lax.Precision.HIGHEST,
        preferred_element_type=jnp.float32) + bl_ref[...]


def _tc_fused_body(sum_ref, cnt_ref, xr_ref, wl_ref, g_ref, bt_ref, o_ref):
    summed = sum_ref[0] + sum_ref[1]
    counts = cnt_ref[0, :, 0:1] + cnt_ref[1, :, 0:1]
    mean = summed / jnp.maximum(counts, 1.0)
    out = lax.dot_general(mean, wl_ref[...],
                          (((1,), (1,)), ((), ())),
                          precision=lax.Precision.HIGHEST,
                          preferred_element_type=jnp.float32) + xr_ref[...]
    nrm = jnp.sqrt(jnp.sum(out * out, axis=-1, keepdims=True))
    out = out / jnp.maximum(nrm, 1e-12)
    mu = jnp.sum(out, axis=0, keepdims=True) * (1.0 / N_NODES)
    ex2 = jnp.sum(out * out, axis=0, keepdims=True) * (1.0 / N_NODES)
    var = ex2 - mu * mu
    out = (out - mu) / jnp.sqrt(var + 1e-5) * g_ref[...] + bt_ref[...]
    o_ref[...] = jnp.maximum(out, 0.0)


@jax.jit
def _tc_xr(x, W_r, b_l):
    return pl.pallas_call(
        _tc_xr_body,
        out_shape=jax.ShapeDtypeStruct((N_NODES, D), jnp.float32),
    )(x, W_r, b_l.reshape(1, D))


@jax.jit
def _tc_finish(sum_parts, cnt_parts, xr, W_l, gamma, beta):
    return pl.pallas_call(
        _tc_fused_body,
        out_shape=jax.ShapeDtypeStruct((N_NODES, D), jnp.float32),
    )(sum_parts, cnt_parts, xr, W_l, gamma.reshape(1, D),
      beta.reshape(1, D))


def kernel(x, edge_index, W_l, b_l, W_r, gamma, beta):
    idx2d = jnp.transpose(
        edge_index.astype(jnp.int32).reshape(2, N_CHUNKS, CHUNK), (1, 0, 2))
    xr = _tc_xr(x, W_r, b_l)
    sum_parts, cnt_parts = _sc_aggregate(idx2d, x)
    return _tc_finish(sum_parts, cnt_parts, xr, W_l, gamma, beta)


# speedup vs baseline: 1.3268x; 1.0796x over previous
import jax
import jax.numpy as jnp
from jax import lax
from jax.experimental import pallas as pl
from jax.experimental.pallas import tpu as pltpu
from jax.experimental.pallas import tpu_sc as plsc

N_NODES = 10000
N_EDGES = 320000
D = 128
CNT = 16
CHUNK = 80
N_CHUNKS = N_EDGES // CHUNK
NC = 2
NS = 16
NW = NC * NS
TILE_ITERS = N_CHUNKS // NW
STRIPE = 624
LAST_STRIPE = N_NODES - 15 * STRIPE
NBUF = 3


def _sc_body(idx_hbm, x_hbm, sum_out, cnt_out,
             idx_v, rows_v, ones_v, zbuf, zcnt, acc_sh, cnt_sh, sems,
             isems, zsem):
    c = lax.axis_index("c")
    s = lax.axis_index("s")
    wid = s * NC + c

    zrow = jnp.zeros((16,), jnp.float32)
    orow = jnp.ones((16,), jnp.float32)
    for r in range(16):
        for g in range(D // 16):
            zbuf[r, pl.ds(g * 16, 16)] = zrow
        zcnt[r, pl.ds(0, 16)] = zrow
    for r in range(CHUNK):
        ones_v[r, pl.ds(0, 16)] = orow

    n_z = jnp.where(s == NS - 1, LAST_STRIPE // 16, STRIPE // 16)

    def zero_fire(j, base):
        pltpu.async_copy(zbuf, acc_sh.at[pl.ds(base + j * 16, 16)], zsem)
        pltpu.async_copy(zcnt, cnt_sh.at[pl.ds(base + j * 16, 16)], zsem)
        return base

    def zero_drain(j, base):
        pltpu.make_async_copy(zbuf, acc_sh.at[pl.ds(base, 16)], zsem).wait()
        pltpu.make_async_copy(zcnt, cnt_sh.at[pl.ds(base, 16)], zsem).wait()
        return base

    lax.fori_loop(0, n_z, zero_fire, s * STRIPE, unroll=False)
    lax.fori_loop(0, n_z, zero_drain, s * STRIPE, unroll=False)

    plsc.subcore_barrier()

    def valid(u):
        return jnp.logical_and(u >= 0, u < TILE_ITERS)

    def idx_issue(t, m):
        chunk = t * NW + wid
        pltpu.async_copy(idx_hbm.at[chunk], idx_v.at[m], isems.at[m])

    def gather_issue(t, k, m):
        chunk = t * NW + wid
        pltpu.make_async_copy(idx_hbm.at[chunk], idx_v.at[m],
                              isems.at[m]).wait()
        pltpu.async_copy(x_hbm.at[idx_v.at[m, 0]], rows_v.at[k], sems.at[k])

    def scatter_issue(k, m):
        pltpu.make_async_copy(x_hbm.at[idx_v.at[m, 0]], rows_v.at[k],
                              sems.at[k]).wait()
        pltpu.async_copy(rows_v.at[k], acc_sh.at[idx_v.at[m, 1]], sems.at[k],
                         add=True)
        pltpu.async_copy(ones_v, cnt_sh.at[idx_v.at[m, 1]], sems.at[k],
                         add=True)

    def drain(k, m):
        pltpu.make_async_copy(rows_v.at[k], acc_sh.at[idx_v.at[m, 1]],
                              sems.at[k]).wait()
        pltpu.make_async_copy(ones_v, cnt_sh.at[idx_v.at[m, 1]],
                              sems.at[k]).wait()

    def pipe_step(j, carry):
        for k6 in range(2 * NBUF):
            u = j * (2 * NBUF) + k6
            t = u - NBUF
            k = k6 % NBUF

            @pl.when(valid(t - 3))
            def _(k=k, k6=k6):
                drain(k, k6)

            @pl.when(valid(t))
            def _(t=t, k=k, k6=k6):
                gather_issue(t, k, (k6 + 3) % (2 * NBUF))

            @pl.when(valid(t - 2))
            def _(k=k, k6=k6):
                scatter_issue((k + 1) % NBUF, (k6 + 1) % (2 * NBUF))

            @pl.when(valid(t + 3))
            def _(t=t, k6=k6):
                idx_issue(t + 3, k6)
        return carry

    n_slots = TILE_ITERS + 2 * NBUF
    lax.fori_loop(0, (n_slots + 2 * NBUF - 1) // (2 * NBUF), pipe_step, 0,
                  unroll=False)

    plsc.subcore_barrier()

    @pl.when(s < NS - 1)
    def _():
        base = s * STRIPE
        pltpu.sync_copy(acc_sh.at[pl.ds(base, STRIPE)],
                        sum_out.at[c, pl.ds(base, STRIPE)])
        pltpu.sync_copy(cnt_sh.at[pl.ds(base, STRIPE)],
                        cnt_out.at[c, pl.ds(base, STRIPE)])

    @pl.when(s == NS - 1)
    def _():
        base = (NS - 1) * STRIPE
        pltpu.sync_copy(acc_sh.at[pl.ds(base, LAST_STRIPE)],
                        sum_out.at[c, pl.ds(base, LAST_STRIPE)])
        pltpu.sync_copy(cnt_sh.at[pl.ds(base, LAST_STRIPE)],
                        cnt_out.at[c, pl.ds(base, LAST_STRIPE)])


@jax.jit
def _sc_aggregate(idx2d, x):
    mesh = plsc.VectorSubcoreMesh(core_axis_name="c", subcore_axis_name="s")
    return pl.kernel(
        _sc_body,
        out_type=[
            jax.ShapeDtypeStruct((NC, N_NODES, D), jnp.float32),
            jax.ShapeDtypeStruct((NC, N_NODES, CNT), jnp.float32),
        ],
        mesh=mesh,
        scratch_types=[
            pltpu.VMEM((2 * NBUF, 2, CHUNK), jnp.int32),
            pltpu.VMEM((NBUF, CHUNK, D), jnp.float32),
            pltpu.VMEM((CHUNK, CNT), jnp.float32),
            pltpu.VMEM((16, D), jnp.float32),
            pltpu.VMEM((16, CNT), jnp.float32),
            pltpu.VMEM_SHARED((N_NODES, D), jnp.float32),
            pltpu.VMEM_SHARED((N_NODES, CNT), jnp.float32),
            pltpu.SemaphoreType.DMA((NBUF,)),
            pltpu.SemaphoreType.DMA((2 * NBUF,)),
            pltpu.SemaphoreType.DMA,
        ],
        compiler_params=pltpu.CompilerParams(use_tc_tiling_on_sc=False),
    )(idx2d, x)


def _tc_xr_body(x_ref, wr_ref, bl_ref, xr_ref):
    xr_ref[...] = lax.dot_general(
        x_ref[...], wr_ref[...],
        (((1,), (1,)), ((), ())),
        precision=lax.Precision.HIGHEST,
        preferred_element_type=jnp.float32) + bl_ref[...]


def _tc_fused_body(sum_ref, cnt_ref, xr_ref, wl_ref, g_ref, bt_ref, o_ref):
    summed = sum_ref[0] + sum_ref[1]
    counts = cnt_ref[0, :, 0:1] + cnt_ref[1, :, 0:1]
    mean = summed / jnp.maximum(counts, 1.0)
    out = lax.dot_general(mean, wl_ref[...],
                          (((1,), (1,)), ((), ())),
                          precision=lax.Precision.HIGHEST,
                          preferred_element_type=jnp.float32) + xr_ref[...]
    nrm = jnp.sqrt(jnp.sum(out * out, axis=-1, keepdims=True))
    out = out / jnp.maximum(nrm, 1e-12)
    mu = jnp.sum(out, axis=0, keepdims=True) * (1.0 / N_NODES)
    ex2 = jnp.sum(out * out, axis=0, keepdims=True) * (1.0 / N_NODES)
    var = ex2 - mu * mu
    out = (out - mu) / jnp.sqrt(var + 1e-5) * g_ref[...] + bt_ref[...]
    o_ref[...] = jnp.maximum(out, 0.0)


@jax.jit
def _tc_xr(x, W_r, b_l):
    return pl.pallas_call(
        _tc_xr_body,
        out_shape=jax.ShapeDtypeStruct((N_NODES, D), jnp.float32),
    )(x, W_r, b_l.reshape(1, D))


@jax.jit
def _tc_finish(sum_parts, cnt_parts, xr, W_l, gamma, beta):
    return pl.pallas_call(
        _tc_fused_body,
        out_shape=jax.ShapeDtypeStruct((N_NODES, D), jnp.float32),
    )(sum_parts, cnt_parts, xr, W_l, gamma.reshape(1, D),
      beta.reshape(1, D))


def kernel(x, edge_index, W_l, b_l, W_r, gamma, beta):
    idx2d = jnp.transpose(
        edge_index.astype(jnp.int32).reshape(2, N_CHUNKS, CHUNK), (1, 0, 2))
    xr = _tc_xr(x, W_r, b_l)
    sum_parts, cnt_parts = _sc_aggregate(idx2d, x)
    return _tc_finish(sum_parts, cnt_parts, xr, W_l, gamma, beta)
